# baseline (speedup 1.0000x reference)
import functools

import numpy as np
import jax
import jax.numpy as jnp
from jax import lax
from jax.experimental import pallas as pl
from jax.experimental.pallas import tpu as pltpu
from jax.experimental.pallas import tpu_sc as plsc

_FEATURE_DIM = 2
_NUM_LVL = 16
_OUT_DIM = _NUM_LVL * _FEATURE_DIM
_MAX_RES = 2048.0
_MIN_RES = 16.0
_HASH_POWER = 19
_MAX_ENTRY = 2 ** _HASH_POWER
_PRIMES = (3367900313, 2654435761, 805459861)
_BGROW = np.exp((np.log(_MAX_RES) - np.log(_MIN_RES)) / (_NUM_LVL - 1))
_RESOLUTIONS = [float(np.floor(_MIN_RES * _BGROW ** i)) for i in range(_NUM_LVL)]
_TABLE_SIZES = [int(min(r ** 3, _MAX_ENTRY)) for r in _RESOLUTIONS]

_N_PTS = 524288
_NC = 2
_NS = 16
_NW = _NC * _NS
_PW = _N_PTS // _NW
_T = 256
_G = _T // 16
_G8 = _T // 8
_NT = _PW // _T

_STAGED_LVLS = (0, 1, 2)
_STAGE_OFF = []
_off = 0
for _l in _STAGED_LVLS:
    _STAGE_OFF.append(_off)
    _off += 2 * _TABLE_SIZES[_l]
_STAGE_WORDS = _off
_BIG_LVLS = tuple(l for l in range(_NUM_LVL) if l not in _STAGED_LVLS)

_P32 = tuple(int(np.int32(np.uint32(p & 0xFFFFFFFF))) for p in _PRIMES)


def _grid_body(x_hbm, *rest):
    tables = rest[:_NUM_LVL]
    out_hbm = rest[_NUM_LVL]
    (xt0, xt1, xt2, idxb0, idxb1, wb0, wb1, rows0, rows1, outt, smalltab,
     sem0, sem1, sem2, sem3) = rest[_NUM_LVL + 1:]
    idxbs = (idxb0, idxb1)
    wbs = (wb0, wb1)
    rowss = (rows0, rows1)
    sems = ((sem0, sem2), (sem1, sem3))

    cid = lax.axis_index("c")
    sid = lax.axis_index("s")
    wid = sid * _NC + cid
    iota = lax.broadcasted_iota(jnp.int32, (16,), 0)
    ihalf = lax.shift_right_logical(iota, jnp.int32(1))
    ibit = iota & jnp.int32(1)

    def make_idx_loop(lvl, idxb, wb):
        res = _RESOLUTIONS[lvl]
        scale = jnp.float32(np.float32(res - 1.0))
        hi = jnp.float32(np.float32(res - 1.0001))
        dense = _TABLE_SIZES[lvl] != _MAX_ENTRY

        def idx_body(g, c):
            g16 = g * jnp.int32(16)
            fx = xt0[pl.ds(g16, 16)]
            fy = xt1[pl.ds(g16, 16)]
            fz = xt2[pl.ds(g16, 16)]
            zero = jnp.float32(0.0)
            cx = jnp.minimum(jnp.maximum(fx * scale, zero), hi)
            cy = jnp.minimum(jnp.maximum(fy * scale, zero), hi)
            cz = jnp.minimum(jnp.maximum(fz * scale, zero), hi)
            ix = cx.astype(jnp.int32)
            iy = cy.astype(jnp.int32)
            iz = cz.astype(jnp.int32)
            dx = cx - ix.astype(jnp.float32)
            dy = cy - iy.astype(jnp.float32)
            dz = cz - iz.astype(jnp.float32)
            mx = jnp.float32(1.0) - dx
            my = jnp.float32(1.0) - dy
            mz = jnp.float32(1.0) - dz
            if dense:
                r = int(res)
                x0 = ix
                x1 = ix + jnp.int32(1)
                y0 = iy * jnp.int32(r)
                y1 = y0 + jnp.int32(r)
                z0 = iz * jnp.int32(r * r)
                z1 = z0 + jnp.int32(r * r)

                def combine(a, b, cc):
                    return a + b + cc
            else:
                p0 = jnp.int32(_P32[0])
                p1 = jnp.int32(_P32[1])
                p2 = jnp.int32(_P32[2])
                mask = jnp.int32(_MAX_ENTRY - 1)
                x0 = ix * p0
                x1 = x0 + p0
                y0 = iy * p1
                y1 = y0 + p1
                z0 = iz * p2
                z1 = z0 + p2

                def combine(a, b, cc):
                    return ((a ^ b) ^ cc) & mask

            w00 = mx * my
            w01 = mx * dy
            w10 = dx * my
            w11 = dx * dy
            xs = (x0, x1)
            ys = (y0, y1)
            zs = (z0, z1)
            wxy = ((w00, w01), (w10, w11))
            wz = (mz, dz)
            pe0 = (g16 + iota) * jnp.int32(2)
            k = 0
            for a in range(2):
                for b in range(2):
                    for c2 in range(2):
                        idxv = combine(xs[a], ys[b], zs[c2])
                        wv = wxy[a][b] * wz[c2]
                        idx2 = idxv * jnp.int32(2)
                        pe = pe0 + jnp.int32(2 * k * _T)
                        po = pe + jnp.int32(1)
                        plsc.store_scatter(idxb, [pe], idx2)
                        plsc.store_scatter(idxb, [po], idx2 + jnp.int32(1))
                        plsc.store_scatter(wb, [pe], wv)
                        plsc.store_scatter(wb, [po], wv)
                        k += 1
            return c

        lax.fori_loop(jnp.int32(0), jnp.int32(_G), idx_body, jnp.int32(0))

    def make_sum_loop(lvl, wb, rows):
        def sum_body(g, c):
            q2 = g * jnp.int32(16)
            acc = jnp.zeros((16,), jnp.float32)
            for k in range(8):
                off = jnp.int32(2 * k * _T) + q2
                wv = wb[pl.ds(off, 16)]
                fv = rows[pl.ds(off, 16)]
                acc = acc + wv * fv
            pos = ((g * jnp.int32(8) + ihalf) * jnp.int32(_OUT_DIM)
                   + (jnp.int32(2 * lvl) + ibit))
            plsc.store_scatter(outt, [pos], acc)
            return c

        lax.fori_loop(jnp.int32(0), jnp.int32(_G8), sum_body, jnp.int32(0))

    def make_small_level(si, lvl):
        res = _RESOLUTIONS[lvl]
        scale = jnp.float32(np.float32(res - 1.0))
        hi = jnp.float32(np.float32(res - 1.0001))
        r = int(res)
        off = _STAGE_OFF[si]

        def body(g, c):
            g16 = g * jnp.int32(16)
            fx = xt0[pl.ds(g16, 16)]
            fy = xt1[pl.ds(g16, 16)]
            fz = xt2[pl.ds(g16, 16)]
            zero = jnp.float32(0.0)
            cx = jnp.minimum(jnp.maximum(fx * scale, zero), hi)
            cy = jnp.minimum(jnp.maximum(fy * scale, zero), hi)
            cz = jnp.minimum(jnp.maximum(fz * scale, zero), hi)
            ix = cx.astype(jnp.int32)
            iy = cy.astype(jnp.int32)
            iz = cz.astype(jnp.int32)
            dx = cx - ix.astype(jnp.float32)
            dy = cy - iy.astype(jnp.float32)
            dz = cz - iz.astype(jnp.float32)
            mx = jnp.float32(1.0) - dx
            my = jnp.float32(1.0) - dy
            mz = jnp.float32(1.0) - dz
            x0 = ix * jnp.int32(2)
            x1 = x0 + jnp.int32(2)
            y0 = iy * jnp.int32(2 * r)
            y1 = y0 + jnp.int32(2 * r)
            z0 = iz * jnp.int32(2 * r * r) + jnp.int32(off)
            z1 = z0 + jnp.int32(2 * r * r)
            w00 = mx * my
            w01 = mx * dy
            w10 = dx * my
            w11 = dx * dy
            xs = (x0, x1)
            ys = (y0, y1)
            zs = (z0, z1)
            wxy = ((w00, w01), (w10, w11))
            wz = (mz, dz)
            acc0 = jnp.zeros((16,), jnp.float32)
            acc1 = jnp.zeros((16,), jnp.float32)
            for a in range(2):
                for b in range(2):
                    for c2 in range(2):
                        pos = xs[a] + ys[b] + zs[c2]
                        wv = wxy[a][b] * wz[c2]
                        f0 = plsc.load_gather(smalltab, [pos])
                        f1 = plsc.load_gather(smalltab,
                                              [pos + jnp.int32(1)])
                        acc0 = acc0 + wv * f0
                        acc1 = acc1 + wv * f1
            opos = (g16 + iota) * jnp.int32(_OUT_DIM) + jnp.int32(2 * lvl)
            plsc.store_scatter(outt, [opos], acc0)
            plsc.store_scatter(outt, [opos + jnp.int32(1)], acc1)
            return c

        lax.fori_loop(jnp.int32(0), jnp.int32(_G), body, jnp.int32(0))

    for si, lvl in enumerate(_STAGED_LVLS):
        pltpu.sync_copy(
            tables[lvl],
            smalltab.at[pl.ds(_STAGE_OFF[si], 2 * _TABLE_SIZES[lvl])])

    def tile_body(t, carry):
        base = wid * jnp.int32(_PW) + t * jnp.int32(_T)
        pltpu.sync_copy(x_hbm.at[pl.ds(base, _T)], xt0)
        pltpu.sync_copy(x_hbm.at[pl.ds(jnp.int32(_N_PTS) + base, _T)], xt1)
        pltpu.sync_copy(x_hbm.at[pl.ds(jnp.int32(2 * _N_PTS) + base, _T)],
                        xt2)
        prev_cp = None
        prev_lvl = None
        half = 8 * _T
        for j, lvl in enumerate(_BIG_LVLS):
            cur = j % 2
            make_idx_loop(lvl, idxbs[cur], wbs[cur])
            cp = (
                pltpu.async_copy(
                    tables[lvl].at[idxbs[cur].at[pl.ds(0, half)]],
                    rowss[cur].at[pl.ds(0, half)], sems[cur][0]),
                pltpu.async_copy(
                    tables[lvl].at[idxbs[cur].at[pl.ds(half, half)]],
                    rowss[cur].at[pl.ds(half, half)], sems[cur][1]),
            )
            if j == 0:
                for si, slvl in enumerate(_STAGED_LVLS):
                    make_small_level(si, slvl)
            else:
                prev_cp[0].wait()
                prev_cp[1].wait()
                make_sum_loop(prev_lvl, wbs[1 - cur], rowss[1 - cur])
            prev_cp = cp
            prev_lvl = lvl
        prev_cp[0].wait()
        prev_cp[1].wait()
        make_sum_loop(prev_lvl, wbs[(len(_BIG_LVLS) - 1) % 2],
                      rowss[(len(_BIG_LVLS) - 1) % 2])
        pltpu.sync_copy(outt, out_hbm.at[pl.ds(base * jnp.int32(_OUT_DIM),
                                               _T * _OUT_DIM)])
        return carry

    lax.fori_loop(jnp.int32(0), jnp.int32(_NT), tile_body, jnp.int32(0))


@functools.lru_cache(maxsize=1)
def _build_grid_kernel():
    mesh = plsc.VectorSubcoreMesh(core_axis_name="c", subcore_axis_name="s")
    return functools.partial(
        pl.kernel,
        mesh=mesh,
        out_type=jax.ShapeDtypeStruct((_N_PTS * _OUT_DIM,), jnp.float32),
        compiler_params=pltpu.CompilerParams(needs_layout_passes=False),
        scratch_types=[
            pltpu.VMEM((_T,), jnp.float32),
            pltpu.VMEM((_T,), jnp.float32),
            pltpu.VMEM((_T,), jnp.float32),
            pltpu.VMEM((16 * _T,), jnp.int32),
            pltpu.VMEM((16 * _T,), jnp.int32),
            pltpu.VMEM((16 * _T,), jnp.float32),
            pltpu.VMEM((16 * _T,), jnp.float32),
            pltpu.VMEM((16 * _T,), jnp.float32),
            pltpu.VMEM((16 * _T,), jnp.float32),
            pltpu.VMEM((_T * _OUT_DIM,), jnp.float32),
            pltpu.VMEM((_STAGE_WORDS,), jnp.float32),
            pltpu.SemaphoreType.DMA,
            pltpu.SemaphoreType.DMA,
            pltpu.SemaphoreType.DMA,
            pltpu.SemaphoreType.DMA,
        ],
    )(_grid_body)


def kernel(x, table_0, table_1, table_2, table_3, table_4, table_5, table_6,
           table_7, table_8, table_9, table_10, table_11, table_12,
           table_13, table_14, table_15):
    tables = (table_0, table_1, table_2, table_3, table_4, table_5, table_6,
              table_7, table_8, table_9, table_10, table_11, table_12,
              table_13, table_14, table_15)
    x_flat = jnp.transpose(x).reshape(-1)
    tabs_flat = tuple(t.reshape(-1) for t in tables)
    out = _build_grid_kernel()(x_flat, *tabs_flat)
    return out.reshape(_N_PTS, _OUT_DIM)


# speedup vs baseline: 1.2864x; 1.2864x over previous
import functools

import numpy as np
import jax
import jax.numpy as jnp
from jax import lax
from jax.experimental import pallas as pl
from jax.experimental.pallas import tpu as pltpu
from jax.experimental.pallas import tpu_sc as plsc

_FEATURE_DIM = 2
_NUM_LVL = 16
_OUT_DIM = _NUM_LVL * _FEATURE_DIM
_MAX_RES = 2048.0
_MIN_RES = 16.0
_HASH_POWER = 19
_MAX_ENTRY = 2 ** _HASH_POWER
_PRIMES = (3367900313, 2654435761, 805459861)
_BGROW = np.exp((np.log(_MAX_RES) - np.log(_MIN_RES)) / (_NUM_LVL - 1))
_RESOLUTIONS = [float(np.floor(_MIN_RES * _BGROW ** i)) for i in range(_NUM_LVL)]
_TABLE_SIZES = [int(min(r ** 3, _MAX_ENTRY)) for r in _RESOLUTIONS]

_N_PTS = 524288
_NC = 2
_NS = 16
_NW = _NC * _NS
_PW = _N_PTS // _NW
_T = 256
_G = _T // 16
_ND = 8 * _T
_NT = _PW // _T

_STAGED_LVLS = (0, 1)
_STAGE_OFF = []
_off = 0
for _l in _STAGED_LVLS:
    _STAGE_OFF.append(_off)
    _off += 2 * _TABLE_SIZES[_l]
_STAGE_WORDS = _off
_BIG_LVLS = tuple(l for l in range(_NUM_LVL) if l not in _STAGED_LVLS)

_P32 = tuple(int(np.int32(np.uint32(p & 0xFFFFFFFF))) for p in _PRIMES)


def _grid_body(x_hbm, *rest):
    tables = rest[:_NUM_LVL]
    out_hbm = rest[_NUM_LVL]
    (xt0, xt1, xt2, idxb0, idxb1, colb0, colb1, wb0, wb1, rows0, rows1,
     outt, smalltab, sem0, sem1) = rest[_NUM_LVL + 1:]
    idxbs = (idxb0, idxb1)
    colbs = (colb0, colb1)
    wbs = (wb0, wb1)
    rowss = (rows0, rows1)
    sems = (sem0, sem1)

    cid = lax.axis_index("c")
    sid = lax.axis_index("s")
    wid = sid * _NC + cid
    iota = lax.broadcasted_iota(jnp.int32, (16,), 0)

    def coords(g16, scale, hi):
        fx = xt0[pl.ds(g16, 16)]
        fy = xt1[pl.ds(g16, 16)]
        fz = xt2[pl.ds(g16, 16)]
        zero = jnp.float32(0.0)
        cx = jnp.minimum(jnp.maximum(fx * scale, zero), hi)
        cy = jnp.minimum(jnp.maximum(fy * scale, zero), hi)
        cz = jnp.minimum(jnp.maximum(fz * scale, zero), hi)
        ix = cx.astype(jnp.int32)
        iy = cy.astype(jnp.int32)
        iz = cz.astype(jnp.int32)
        dx = cx - ix.astype(jnp.float32)
        dy = cy - iy.astype(jnp.float32)
        dz = cz - iz.astype(jnp.float32)
        mx = jnp.float32(1.0) - dx
        my = jnp.float32(1.0) - dy
        mz = jnp.float32(1.0) - dz
        w00 = mx * my
        w01 = mx * dy
        w10 = dx * my
        w11 = dx * dy
        wxy = ((w00, w01), (w10, w11))
        return ix, iy, iz, wxy, (mz, dz)

    def make_idx_loop(lvl, idxb, colb, wb):
        res = _RESOLUTIONS[lvl]
        scale = jnp.float32(np.float32(res - 1.0))
        hi = jnp.float32(np.float32(res - 1.0001))
        dense = _TABLE_SIZES[lvl] != _MAX_ENTRY

        def idx_body(g, c):
            g16 = g * jnp.int32(16)
            ix, iy, iz, wxy, wz = coords(g16, scale, hi)
            if dense:
                r = int(res)
                x0 = ix
                x1 = ix + jnp.int32(1)
                y0 = iy * jnp.int32(r)
                y1 = y0 + jnp.int32(r)
                z0 = iz * jnp.int32(r * r)
                z1 = z0 + jnp.int32(r * r)

                def combine(a, b, cc):
                    return a + b + cc
            else:
                p0 = jnp.int32(_P32[0])
                p1 = jnp.int32(_P32[1])
                p2 = jnp.int32(_P32[2])
                mask = jnp.int32(_MAX_ENTRY - 1)
                x0 = ix * p0
                x1 = x0 + p0
                y0 = iy * p1
                y1 = y0 + p1
                z0 = iz * p2
                z1 = z0 + p2

                def combine(a, b, cc):
                    return ((a ^ b) ^ cc) & mask

            xs = (x0, x1)
            ys = (y0, y1)
            zs = (z0, z1)
            wz0, wz1 = wz
            k = 0
            for a in range(2):
                for b in range(2):
                    for c2 in range(2):
                        h = combine(xs[a], ys[b], zs[c2])
                        wv = wxy[a][b] * (wz0 if c2 == 0 else wz1)
                        off = jnp.int32(k * _T) + g16
                        idxb[pl.ds(off, 16)] = lax.shift_right_logical(
                            h, jnp.int32(2))
                        colb[pl.ds(off, 16)] = (
                            lax.shift_left(h, jnp.int32(1)) & jnp.int32(6))
                        wb[pl.ds(off, 16)] = wv
                        k += 1
            return c

        lax.fori_loop(jnp.int32(0), jnp.int32(_G), idx_body, jnp.int32(0))

    def make_sum_loop(lvl, colb, wb, rows):
        def sum_body(g, c):
            g16 = g * jnp.int32(16)
            acc0 = jnp.zeros((16,), jnp.float32)
            acc1 = jnp.zeros((16,), jnp.float32)
            for k in range(8):
                off = jnp.int32(k * _T) + g16
                wv = wb[pl.ds(off, 16)]
                col = colb[pl.ds(off, 16)]
                rowv = off + iota
                f0 = plsc.load_gather(rows, [rowv, col])
                f1 = plsc.load_gather(rows, [rowv, col + jnp.int32(1)])
                acc0 = acc0 + wv * f0
                acc1 = acc1 + wv * f1
            opos = (g16 + iota) * jnp.int32(_OUT_DIM) + jnp.int32(2 * lvl)
            plsc.store_scatter(outt, [opos], acc0)
            plsc.store_scatter(outt, [opos + jnp.int32(1)], acc1)
            return c

        lax.fori_loop(jnp.int32(0), jnp.int32(_G), sum_body, jnp.int32(0))

    def make_small_level(si, lvl):
        res = _RESOLUTIONS[lvl]
        scale = jnp.float32(np.float32(res - 1.0))
        hi = jnp.float32(np.float32(res - 1.0001))
        r = int(res)
        off = _STAGE_OFF[si]

        def body(g, c):
            g16 = g * jnp.int32(16)
            ix, iy, iz, wxy, wz = coords(g16, scale, hi)
            x0 = ix * jnp.int32(2)
            x1 = x0 + jnp.int32(2)
            y0 = iy * jnp.int32(2 * r)
            y1 = y0 + jnp.int32(2 * r)
            z0 = iz * jnp.int32(2 * r * r) + jnp.int32(off)
            z1 = z0 + jnp.int32(2 * r * r)
            xs = (x0, x1)
            ys = (y0, y1)
            zs = (z0, z1)
            wz0, wz1 = wz
            acc0 = jnp.zeros((16,), jnp.float32)
            acc1 = jnp.zeros((16,), jnp.float32)
            for a in range(2):
                for b in range(2):
                    for c2 in range(2):
                        pos = xs[a] + ys[b] + zs[c2]
                        wv = wxy[a][b] * (wz0 if c2 == 0 else wz1)
                        f0 = plsc.load_gather(smalltab, [pos])
                        f1 = plsc.load_gather(smalltab,
                                              [pos + jnp.int32(1)])
                        acc0 = acc0 + wv * f0
                        acc1 = acc1 + wv * f1
            opos = (g16 + iota) * jnp.int32(_OUT_DIM) + jnp.int32(2 * lvl)
            plsc.store_scatter(outt, [opos], acc0)
            plsc.store_scatter(outt, [opos + jnp.int32(1)], acc1)
            return c

        lax.fori_loop(jnp.int32(0), jnp.int32(_G), body, jnp.int32(0))

    for si, lvl in enumerate(_STAGED_LVLS):
        pltpu.sync_copy(
            tables[lvl],
            smalltab.at[pl.ds(_STAGE_OFF[si], 2 * _TABLE_SIZES[lvl])])

    def tile_body(t, carry):
        base = wid * jnp.int32(_PW) + t * jnp.int32(_T)
        pltpu.sync_copy(x_hbm.at[pl.ds(base, _T)], xt0)
        pltpu.sync_copy(x_hbm.at[pl.ds(jnp.int32(_N_PTS) + base, _T)], xt1)
        pltpu.sync_copy(x_hbm.at[pl.ds(jnp.int32(2 * _N_PTS) + base, _T)],
                        xt2)
        prev_cp = None
        prev_lvl = None
        for j, lvl in enumerate(_BIG_LVLS):
            cur = j % 2
            make_idx_loop(lvl, idxbs[cur], colbs[cur], wbs[cur])
            cp = pltpu.async_copy(tables[lvl].at[idxbs[cur]], rowss[cur],
                                  sems[cur])
            if j == 0:
                for si, slvl in enumerate(_STAGED_LVLS):
                    make_small_level(si, slvl)
            else:
                prev_cp.wait()
                make_sum_loop(prev_lvl, colbs[1 - cur], wbs[1 - cur],
                              rowss[1 - cur])
            prev_cp = cp
            prev_lvl = lvl
        prev_cp.wait()
        last = (len(_BIG_LVLS) - 1) % 2
        make_sum_loop(prev_lvl, colbs[last], wbs[last], rowss[last])
        pltpu.sync_copy(outt, out_hbm.at[pl.ds(base * jnp.int32(_OUT_DIM),
                                               _T * _OUT_DIM)])
        return carry

    lax.fori_loop(jnp.int32(0), jnp.int32(_NT), tile_body, jnp.int32(0))


@functools.lru_cache(maxsize=1)
def _build_grid_kernel():
    mesh = plsc.VectorSubcoreMesh(core_axis_name="c", subcore_axis_name="s")
    return functools.partial(
        pl.kernel,
        mesh=mesh,
        out_type=jax.ShapeDtypeStruct((_N_PTS * _OUT_DIM,), jnp.float32),
        compiler_params=pltpu.CompilerParams(
            needs_layout_passes=False, use_tc_tiling_on_sc=False),
        scratch_types=[
            pltpu.VMEM((_T,), jnp.float32),
            pltpu.VMEM((_T,), jnp.float32),
            pltpu.VMEM((_T,), jnp.float32),
            pltpu.VMEM((_ND,), jnp.int32),
            pltpu.VMEM((_ND,), jnp.int32),
            pltpu.VMEM((_ND,), jnp.int32),
            pltpu.VMEM((_ND,), jnp.int32),
            pltpu.VMEM((_ND,), jnp.float32),
            pltpu.VMEM((_ND,), jnp.float32),
            pltpu.VMEM((_ND, 8), jnp.float32),
            pltpu.VMEM((_ND, 8), jnp.float32),
            pltpu.VMEM((_T * _OUT_DIM,), jnp.float32),
            pltpu.VMEM((_STAGE_WORDS,), jnp.float32),
            pltpu.SemaphoreType.DMA,
            pltpu.SemaphoreType.DMA,
        ],
    )(_grid_body)


def kernel(x, table_0, table_1, table_2, table_3, table_4, table_5, table_6,
           table_7, table_8, table_9, table_10, table_11, table_12,
           table_13, table_14, table_15):
    tables = (table_0, table_1, table_2, table_3, table_4, table_5, table_6,
              table_7, table_8, table_9, table_10, table_11, table_12,
              table_13, table_14, table_15)
    x_flat = jnp.transpose(x).reshape(-1)
    tabs = tuple(
        t.reshape(-1) if lvl in _STAGED_LVLS
        else t.reshape(_TABLE_SIZES[lvl] * 2 // 8, 8)
        for lvl, t in enumerate(tables))
    out = _build_grid_kernel()(x_flat, *tabs)
    return out.reshape(_N_PTS, _OUT_DIM)
